# Initial kernel scaffold; baseline (speedup 1.0000x reference)
import functools

import jax
import jax.numpy as jnp
from jax import lax
from jax.experimental import pallas as pl
from jax.experimental.pallas import tpu as pltpu
from jax.experimental.pallas import tpu_sc as plsc

N = 10000
NP = 10016
E = 320000
D = 128
NG = 64
EU = E // 128
NC, NS = 2, 16
RS = NP // NS
UD = -(-EU // (NC * NS))
UA = -(-EU // NS)
RB = NP // 4
HCOL = D // NC

_mesh = plsc.VectorSubcoreMesh(core_axis_name="c", subcore_axis_name="s")


def _sc_degree(edges3, zeros, ones16):
    @functools.partial(
        pl.kernel,
        out_type=jax.ShapeDtypeStruct((NC, NP, 16), jnp.float32),
        mesh=_mesh,
        scratch_types=[
            pltpu.VMEM_SHARED((NP, 16), jnp.float32),
            pltpu.VMEM((UD, 128), jnp.int32),
            pltpu.VMEM((128, 16), jnp.float32),
        ],
    )
    def k(edges_hbm, zeros_hbm, ones_hbm, out_hbm, acc, idx_d, ones_v):
        c = lax.axis_index("c")
        s = lax.axis_index("s")
        w = c * NS + s
        r0 = s * RS
        pltpu.sync_copy(zeros_hbm.at[:, pl.ds(0, 16)], acc.at[pl.ds(r0, RS)])
        pltpu.sync_copy(ones_hbm, ones_v)
        start = (EU * w) // (NC * NS)
        cnt = (EU * (w + 1)) // (NC * NS) - start
        pltpu.sync_copy(edges_hbm.at[1, pl.ds(start, UD), :], idx_d)
        plsc.subcore_barrier()

        @pl.loop(0, cnt)
        def _(u):
            pltpu.sync_copy(ones_v, acc.at[idx_d.at[u]], add=True)

        plsc.subcore_barrier()
        pltpu.sync_copy(acc.at[pl.ds(r0, RS)], out_hbm.at[c, pl.ds(r0, RS), :])

    return k(edges3, zeros, ones16)


def _sc_aggregate(hs, edges3, zeros):
    @functools.partial(
        pl.kernel,
        out_type=jax.ShapeDtypeStruct((NP, D), jnp.float32),
        mesh=_mesh,
        scratch_types=[
            pltpu.VMEM_SHARED((NP, HCOL), jnp.float32),
            pltpu.VMEM_SHARED((NP, HCOL), jnp.float32),
            pltpu.VMEM((UA, 128), jnp.int32),
            pltpu.VMEM((UA, 128), jnp.int32),
            pltpu.VMEM((128, HCOL), jnp.float32),
        ],
    )
    def k(hs_hbm, edges_hbm, zeros_hbm, out_hbm, table, acc, idx_s, idx_d, rows):
        c = lax.axis_index("c")
        s = lax.axis_index("s")
        r0 = s * RS
        pltpu.sync_copy(hs_hbm.at[pl.ds(r0, RS), pl.ds(c * HCOL, HCOL)],
                        table.at[pl.ds(r0, RS)])
        pltpu.sync_copy(zeros_hbm, acc.at[pl.ds(r0, RS)])
        start = (EU * s) // NS
        cnt = (EU * (s + 1)) // NS - start
        pltpu.sync_copy(edges_hbm.at[0, pl.ds(start, UA), :], idx_s)
        pltpu.sync_copy(edges_hbm.at[1, pl.ds(start, UA), :], idx_d)
        plsc.subcore_barrier()

        @pl.loop(0, cnt)
        def _(u):
            pltpu.sync_copy(table.at[idx_s.at[u]], rows)
            pltpu.sync_copy(rows, acc.at[idx_d.at[u]], add=True)

        plsc.subcore_barrier()
        pltpu.sync_copy(acc.at[pl.ds(r0, RS)],
                        out_hbm.at[pl.ds(r0, RS), pl.ds(c * HCOL, HCOL)])

    return k(hs, edges3, zeros)


def _tc_matmul(xp, W):
    def body(x_ref, w_ref, o_ref):
        o_ref[...] = jnp.dot(x_ref[...], w_ref[...],
                             preferred_element_type=jnp.float32)

    return pl.pallas_call(
        body,
        grid=(NP // RB,),
        in_specs=[pl.BlockSpec((RB, D), lambda i: (i, 0)),
                  pl.BlockSpec((D, D), lambda i: (0, 0))],
        out_specs=pl.BlockSpec((RB, D), lambda i: (i, 0)),
        out_shape=jax.ShapeDtypeStruct((NP, D), jnp.float32),
    )(xp, W)


def _tc_dinv_scale(deg2, h1):
    def body(d_ref, h_ref, hs_ref, db_ref):
        d = d_ref[0, :, 0:1] + d_ref[1, :, 0:1] + 1.0
        dinv = lax.rsqrt(d)
        hs_ref[...] = dinv * h_ref[...]
        db_ref[...] = jnp.broadcast_to(dinv, h_ref.shape)

    return pl.pallas_call(
        body,
        grid=(NP // RB,),
        in_specs=[pl.BlockSpec((NC, RB, 16), lambda i: (0, i, 0)),
                  pl.BlockSpec((RB, D), lambda i: (i, 0))],
        out_specs=[pl.BlockSpec((RB, D), lambda i: (i, 0)),
                   pl.BlockSpec((RB, D), lambda i: (i, 0))],
        out_shape=[jax.ShapeDtypeStruct((NP, D), jnp.float32),
                   jax.ShapeDtypeStruct((NP, D), jnp.float32)],
    )(deg2, h1)


def _tc_layer_out_matmul(agg, hs, dinvb, b_row, W_next):
    def body(a_ref, hs_ref, db_ref, b_ref, w_ref, o_ref):
        z = jnp.maximum(db_ref[...] * (a_ref[...] + hs_ref[...]) + b_ref[...], 0.0)
        o_ref[...] = db_ref[...] * jnp.dot(z, w_ref[...],
                                           preferred_element_type=jnp.float32)

    return pl.pallas_call(
        body,
        grid=(NP // RB,),
        in_specs=[pl.BlockSpec((RB, D), lambda i: (i, 0)),
                  pl.BlockSpec((RB, D), lambda i: (i, 0)),
                  pl.BlockSpec((RB, D), lambda i: (i, 0)),
                  pl.BlockSpec((1, D), lambda i: (0, 0)),
                  pl.BlockSpec((D, D), lambda i: (0, 0))],
        out_specs=pl.BlockSpec((RB, D), lambda i: (i, 0)),
        out_shape=jax.ShapeDtypeStruct((NP, D), jnp.float32),
    )(agg, hs, dinvb, b_row, W_next)


def _tc_layer_out(agg, hs, dinvb, b_row):
    def body(a_ref, hs_ref, db_ref, b_ref, o_ref):
        o_ref[...] = jnp.maximum(
            db_ref[...] * (a_ref[...] + hs_ref[...]) + b_ref[...], 0.0)

    return pl.pallas_call(
        body,
        grid=(NP // RB,),
        in_specs=[pl.BlockSpec((RB, D), lambda i: (i, 0)),
                  pl.BlockSpec((RB, D), lambda i: (i, 0)),
                  pl.BlockSpec((RB, D), lambda i: (i, 0)),
                  pl.BlockSpec((1, D), lambda i: (0, 0))],
        out_specs=pl.BlockSpec((RB, D), lambda i: (i, 0)),
        out_shape=jax.ShapeDtypeStruct((NP, D), jnp.float32),
    )(agg, hs, dinvb, b_row)


def _tc_pool_head(z2, batch2, lin_W, lin_b_row):
    iota8 = lax.broadcasted_iota(jnp.int32, (8, D), 0)

    def body(z_ref, b_ref, w_ref, lb_ref, o_ref, pooled_ref):
        ids = b_ref[...]
        bounds = [jnp.sum((ids < g).astype(jnp.int32)) for g in range(NG + 1)]
        for g in range(NG):
            start, end = bounds[g], bounds[g + 1]

            def blk(b, macc):
                rows = z_ref[pl.ds(b * 8, 8), :]
                rid = b * 8 + iota8
                m = jnp.where((rid >= start) & (rid < end), rows, -jnp.inf)
                return jnp.maximum(macc, m)

            macc = lax.fori_loop(start // 8, (end + 7) // 8, blk,
                                 jnp.full((8, D), -jnp.inf, jnp.float32))
            pooled_ref[pl.ds(g, 1), :] = jnp.max(macc, axis=0, keepdims=True)
        o_ref[...] = jnp.dot(pooled_ref[...], w_ref[...],
                             preferred_element_type=jnp.float32) + lb_ref[...]

    return pl.pallas_call(
        body,
        out_shape=jax.ShapeDtypeStruct((NG, 10), jnp.float32),
        scratch_shapes=[pltpu.VMEM((NG, D), jnp.float32)],
    )(z2, batch2, lin_W, lin_b_row)


def kernel(x, edge_index, batch, W1, b1, W2, b2, lin_W, lin_b):
    xp = jnp.pad(x, ((0, NP - N), (0, 0)))
    edges3 = edge_index.reshape(2, EU, 128)
    batch2 = jnp.pad(batch, (0, 80 * 128 - N),
                     constant_values=jnp.int32(2**30)).reshape(80, 128)
    zeros = jnp.zeros((RS, HCOL), jnp.float32)
    ones16 = jnp.ones((128, 16), jnp.float32)

    deg2 = _sc_degree(edges3, zeros, ones16)
    h1 = _tc_matmul(xp, W1)
    hs1, dinvb = _tc_dinv_scale(deg2, h1)
    agg1 = _sc_aggregate(hs1, edges3, zeros)
    hs2 = _tc_layer_out_matmul(agg1, hs1, dinvb, b1.reshape(1, D), W2)
    agg2 = _sc_aggregate(hs2, edges3, zeros)
    z2 = _tc_layer_out(agg2, hs2, dinvb, b2.reshape(1, D))
    return _tc_pool_head(z2, batch2, lin_W, lin_b.reshape(1, 10))


# speedup vs baseline: 21.8089x; 21.8089x over previous
import functools

import jax
import jax.numpy as jnp
from jax import lax
from jax.experimental import pallas as pl
from jax.experimental.pallas import tpu as pltpu
from jax.experimental.pallas import tpu_sc as plsc

N = 10000
NP = 10112
E = 320000
D = 128
NG = 64
NC, NS = 2, 16
EUP = 2560
EP = EUP * 128
UW = EUP // (NC * NS)
RS = NP // NS
RB = NP // 4

_mesh = plsc.VectorSubcoreMesh(core_axis_name="c", subcore_axis_name="s")


def _sc_degree(edges3, zeros16, ones16):
    @functools.partial(
        pl.kernel,
        out_type=jax.ShapeDtypeStruct((NC, NP, 16), jnp.float32),
        mesh=_mesh,
        scratch_types=[
            pltpu.VMEM_SHARED((NP, 16), jnp.float32),
            pltpu.VMEM((UW, 128), jnp.int32),
            pltpu.VMEM((128, 16), jnp.float32),
        ],
    )
    def k(edges_hbm, zeros_hbm, ones_hbm, out_hbm, acc, idx_d, ones_v):
        c = lax.axis_index("c")
        s = lax.axis_index("s")
        w = c * NS + s
        r0 = s * RS
        pltpu.sync_copy(zeros_hbm, acc.at[pl.ds(r0, RS)])
        pltpu.sync_copy(ones_hbm, ones_v)
        e0 = pl.multiple_of(w * UW, 8)
        pltpu.sync_copy(edges_hbm.at[1, pl.ds(e0, UW), :], idx_d)
        plsc.subcore_barrier()

        @pl.loop(0, UW)
        def _(u):
            pltpu.sync_copy(ones_v, acc.at[idx_d.at[u]], add=True)

        plsc.subcore_barrier()
        pltpu.sync_copy(acc.at[pl.ds(r0, RS)], out_hbm.at[c, pl.ds(r0, RS), :])

    return k(edges3, zeros16, ones16)


def _sc_aggregate(hs, edges3):
    @functools.partial(
        pl.kernel,
        out_type=jax.ShapeDtypeStruct((NC, NP, D), jnp.float32),
        mesh=_mesh,
        scratch_types=[
            pltpu.VMEM_SHARED((NP, D), jnp.float32),
            pltpu.VMEM((UW, 128), jnp.int32),
            pltpu.VMEM((UW, 128), jnp.int32),
            pltpu.VMEM((128, D), jnp.float32),
        ],
    )
    def k(hs_hbm, edges_hbm, out_hbm, acc, idx_s, idx_d, rows):
        c = lax.axis_index("c")
        s = lax.axis_index("s")
        w = c * NS + s
        r0 = s * RS

        @pl.loop(0, 128)
        def _(i):
            @pl.loop(0, D, step=16)
            def _(j):
                rows[i, pl.ds(j, 16)] = jnp.zeros((16,), jnp.float32)

        @pl.loop(0, 4)
        def _(i):
            pltpu.sync_copy(rows, acc.at[pl.ds(r0 + i * 128, 128)])

        pltpu.sync_copy(rows.at[pl.ds(0, 120)], acc.at[pl.ds(r0 + 512, 120)])

        e0 = pl.multiple_of(w * UW, 8)
        pltpu.sync_copy(edges_hbm.at[0, pl.ds(e0, UW), :], idx_s)
        pltpu.sync_copy(edges_hbm.at[1, pl.ds(e0, UW), :], idx_d)
        plsc.subcore_barrier()

        @pl.loop(0, UW)
        def _(u):
            pltpu.sync_copy(hs_hbm.at[idx_s.at[u]], rows)
            pltpu.sync_copy(rows, acc.at[idx_d.at[u]], add=True)

        plsc.subcore_barrier()

        @pl.loop(0, 3)
        def _(i):
            pltpu.sync_copy(acc.at[pl.ds(r0 + i * 160, 160)],
                            out_hbm.at[c, pl.ds(r0 + i * 160, 160), :])

        pltpu.sync_copy(acc.at[pl.ds(r0 + 480, 152)],
                        out_hbm.at[c, pl.ds(r0 + 480, 152), :])

    return k(hs, edges3)


def _tc_matmul(xp, W):
    def body(x_ref, w_ref, o_ref):
        o_ref[...] = jnp.dot(x_ref[...], w_ref[...],
                             preferred_element_type=jnp.float32)

    return pl.pallas_call(
        body,
        grid=(NP // RB,),
        in_specs=[pl.BlockSpec((RB, D), lambda i: (i, 0)),
                  pl.BlockSpec((D, D), lambda i: (0, 0))],
        out_specs=pl.BlockSpec((RB, D), lambda i: (i, 0)),
        out_shape=jax.ShapeDtypeStruct((NP, D), jnp.float32),
    )(xp, W)


def _tc_dinv_scale(deg2, h1):
    def body(d_ref, h_ref, hs_ref, db_ref):
        d = d_ref[0, :, 0:1] + d_ref[1, :, 0:1] + 1.0
        dinv = lax.rsqrt(d)
        hs_ref[...] = dinv * h_ref[...]
        db_ref[...] = jnp.broadcast_to(dinv, h_ref.shape)

    return pl.pallas_call(
        body,
        grid=(NP // RB,),
        in_specs=[pl.BlockSpec((NC, RB, 16), lambda i: (0, i, 0)),
                  pl.BlockSpec((RB, D), lambda i: (i, 0))],
        out_specs=[pl.BlockSpec((RB, D), lambda i: (i, 0)),
                   pl.BlockSpec((RB, D), lambda i: (i, 0))],
        out_shape=[jax.ShapeDtypeStruct((NP, D), jnp.float32),
                   jax.ShapeDtypeStruct((NP, D), jnp.float32)],
    )(deg2, h1)


def _tc_layer_out_matmul(agg2, hs, dinvb, b_row, W_next):
    def body(a_ref, hs_ref, db_ref, b_ref, w_ref, o_ref):
        agg = a_ref[0] + a_ref[1]
        z = jnp.maximum(db_ref[...] * (agg + hs_ref[...]) + b_ref[...], 0.0)
        o_ref[...] = db_ref[...] * jnp.dot(z, w_ref[...],
                                           preferred_element_type=jnp.float32)

    return pl.pallas_call(
        body,
        grid=(NP // RB,),
        in_specs=[pl.BlockSpec((NC, RB, D), lambda i: (0, i, 0)),
                  pl.BlockSpec((RB, D), lambda i: (i, 0)),
                  pl.BlockSpec((RB, D), lambda i: (i, 0)),
                  pl.BlockSpec((1, D), lambda i: (0, 0)),
                  pl.BlockSpec((D, D), lambda i: (0, 0))],
        out_specs=pl.BlockSpec((RB, D), lambda i: (i, 0)),
        out_shape=jax.ShapeDtypeStruct((NP, D), jnp.float32),
    )(agg2, hs, dinvb, b_row, W_next)


def _tc_layer_out(agg2, hs, dinvb, b_row):
    def body(a_ref, hs_ref, db_ref, b_ref, o_ref):
        agg = a_ref[0] + a_ref[1]
        o_ref[...] = jnp.maximum(
            db_ref[...] * (agg + hs_ref[...]) + b_ref[...], 0.0)

    return pl.pallas_call(
        body,
        grid=(NP // RB,),
        in_specs=[pl.BlockSpec((NC, RB, D), lambda i: (0, i, 0)),
                  pl.BlockSpec((RB, D), lambda i: (i, 0)),
                  pl.BlockSpec((RB, D), lambda i: (i, 0)),
                  pl.BlockSpec((1, D), lambda i: (0, 0))],
        out_specs=pl.BlockSpec((RB, D), lambda i: (i, 0)),
        out_shape=jax.ShapeDtypeStruct((NP, D), jnp.float32),
    )(agg2, hs, dinvb, b_row)


def _tc_pool_head(z2, batch2, lin_W, lin_b_row):
    def body(z_ref, b_ref, w_ref, lb_ref, o_ref, pooled_ref):
        iota8 = lax.broadcasted_iota(jnp.int32, (8, D), 0)
        ids = b_ref[...]
        bounds = [jnp.sum((ids < g).astype(jnp.int32)) for g in range(NG + 1)]
        for g in range(NG):
            start, end = bounds[g], bounds[g + 1]

            def blk(b, macc):
                rows = z_ref[pl.ds(b * 8, 8), :]
                rid = b * 8 + iota8
                m = jnp.where((rid >= start) & (rid < end), rows, -jnp.inf)
                return jnp.maximum(macc, m)

            macc = lax.fori_loop(start // 8, (end + 7) // 8, blk,
                                 jnp.full((8, D), -jnp.inf, jnp.float32))
            pooled_ref[pl.ds(g, 1), :] = jnp.max(macc, axis=0, keepdims=True)
        o_ref[...] = jnp.dot(pooled_ref[...], w_ref[...],
                             preferred_element_type=jnp.float32) + lb_ref[...]

    return pl.pallas_call(
        body,
        out_shape=jax.ShapeDtypeStruct((NG, 10), jnp.float32),
        scratch_shapes=[pltpu.VMEM((NG, D), jnp.float32)],
    )(z2, batch2, lin_W, lin_b_row)


def kernel(x, edge_index, batch, W1, b1, W2, b2, lin_W, lin_b):
    xp = jnp.pad(x, ((0, NP - N), (0, 0)))
    pad_idx = (N + jnp.arange(EP - E, dtype=jnp.int32) % (NP - N))
    src = jnp.concatenate([edge_index[0], pad_idx])
    dst = jnp.concatenate([edge_index[1], pad_idx])
    edges3 = jnp.stack([src, dst]).reshape(2, EUP, 128)
    batch2 = jnp.pad(batch, (0, 80 * 128 - N),
                     constant_values=jnp.int32(2**30)).reshape(80, 128)
    zeros16 = jnp.zeros((RS, 16), jnp.float32)
    ones16 = jnp.ones((128, 16), jnp.float32)

    deg2 = _sc_degree(edges3, zeros16, ones16)
    h1 = _tc_matmul(xp, W1)
    hs1, dinvb = _tc_dinv_scale(deg2, h1)
    agg1 = _sc_aggregate(hs1, edges3)
    hs2 = _tc_layer_out_matmul(agg1, hs1, dinvb, b1.reshape(1, D), W2)
    agg2 = _sc_aggregate(hs2, edges3)
    z2 = _tc_layer_out(agg2, hs2, dinvb, b2.reshape(1, D))
    return _tc_pool_head(z2, batch2, lin_W, lin_b.reshape(1, 10))
